# baseline (speedup 1.0000x reference)
import jax
import jax.numpy as jnp
from jax import lax
from jax.experimental import pallas as pl
from jax.experimental.pallas import tpu as pltpu, tpu_sc as plsc

_HI = jax.lax.Precision.HIGHEST


def _graph_kernel(x_ref, pos_ref, srcl_ref, dstl_ref,
                  fw1_ref, fb1_ref, fw2_ref, fb2_ref,
                  ew1t_ref, ewlast_ref, eb1_ref, ew2t_ref, eb2_ref,
                  fe_ref, am_ref):
    npg = x_ref.shape[1]
    epg = srcl_ref.shape[2]
    nf = fw2_ref.shape[1]
    xg = x_ref[0]
    posg = pos_ref[0]
    srow = srcl_ref[0]
    drow = dstl_ref[0]

    bf = lambda a: a.astype(jnp.bfloat16).astype(jnp.float32)

    h0 = jnp.maximum(jnp.dot(bf(xg), bf(fw1_ref[...]), precision=_HI,
                             preferred_element_type=jnp.float32) + fb1_ref[0], 0.0)
    fv = jax.nn.sigmoid(jnp.dot(bf(h0), bf(fw2_ref[...]), precision=_HI,
                                preferred_element_type=jnp.float32) + fb2_ref[0])

    rows = jax.lax.broadcasted_iota(jnp.int32, (npg, epg), 0)
    ohs = (rows == srow).astype(jnp.float32)
    ohd = (rows == drow).astype(jnp.float32)

    dn = (((0,), (0,)), ((), ()))
    xsum = jax.lax.dot_general(xg, ohs + ohd, dn, precision=_HI,
                               preferred_element_type=jnp.float32)
    ps = jax.lax.dot_general(posg, ohs, dn, precision=_HI,
                             preferred_element_type=jnp.float32)
    pd = jax.lax.dot_general(posg, ohd, dn, precision=_HI,
                             preferred_element_type=jnp.float32)
    d2 = jnp.sum((ps - pd) ** 2, axis=0, keepdims=True)
    dist = jnp.sqrt(d2)

    h1 = jnp.maximum(jnp.dot(bf(ew1t_ref[...]), bf(xsum), precision=_HI,
                             preferred_element_type=jnp.float32)
                     + bf(ewlast_ref[...]) * bf(dist) + eb1_ref[...], 0.0)
    fe = jax.nn.sigmoid(jnp.dot(bf(ew2t_ref[...]), bf(h1), precision=_HI,
                                preferred_element_type=jnp.float32) + eb2_ref[...])

    eye = (jax.lax.broadcasted_iota(jnp.int32, (nf, nf), 0)
           == jax.lax.broadcasted_iota(jnp.int32, (nf, nf), 1)).astype(jnp.float32)
    fe_ref[0] = jax.lax.dot_general(fe, eye, (((0,), (0,)), ((), ())),
                                    precision=_HI,
                                    preferred_element_type=jnp.float32)
    am_ref[0, 0] = jnp.mean(fv, axis=0)


def _sc_body(fe_hbm, src_hbm, dst_hbm, out_hbm, fe_v, src_v, dst_v, gd_v, out_v):
    b = fe_hbm.shape[0]
    nf = 8
    epg = fe_hbm.shape[1] // nf
    npg = gd_v.shape[0] // 16
    nw = 32
    wid = lax.axis_index("s") * 2 + lax.axis_index("c")
    iota = lax.iota(jnp.int32, 16)
    lane_lt8 = iota < nf
    coloff = jnp.bitwise_and(iota, nf - 1)
    init_gd = jnp.where(lane_lt8, jnp.inf, -jnp.inf)

    def splat(v):
        return jnp.full((16,), v, jnp.int32)

    def do_graph(g):
        pltpu.sync_copy(fe_hbm.at[g], fe_v)
        pltpu.sync_copy(src_hbm.at[g], src_v)
        pltpu.sync_copy(dst_hbm.at[g], dst_v)

        def init_row(i, carry):
            plsc.store_scatter(gd_v, [splat(i * 16) + iota], init_gd)
            return carry
        lax.fori_loop(0, npg, init_row, 0)

        def edge_body(k, carry):
            e16 = k * 16 + iota
            s16 = plsc.load_gather(src_v, [e16])
            d16 = plsc.load_gather(dst_v, [e16])
            for j in range(16):
                fe16 = plsc.load_gather(fe_v, [splat((k * 16 + j) * nf) + coloff])
                for node in (s16[j], d16[j]):
                    rows = splat(node * 16) + iota
                    cur = plsc.load_gather(gd_v, [rows])
                    new = jnp.where(lane_lt8, jnp.minimum(cur, fe16),
                                    jnp.maximum(cur, fe16))
                    plsc.store_scatter(gd_v, [rows], new)
            return carry
        lax.fori_loop(0, epg // 16, edge_body, 0)

        def fix_row(i, acc):
            rows = splat(i * 16) + iota
            v = plsc.load_gather(gd_v, [rows])
            v = jnp.where(jnp.abs(v) < jnp.inf, v, 1.0)
            plsc.store_scatter(gd_v, [rows], v)
            return acc + v
        sums = lax.fori_loop(0, npg, fix_row, jnp.zeros((16,), jnp.float32))
        mean16 = sums / jnp.float32(npg)

        def cyc_body(k, accs):
            e16 = k * 16 + iota
            s16 = plsc.load_gather(src_v, [e16])
            d16 = plsc.load_gather(dst_v, [e16])
            out = []
            for f in range(nf):
                fef = plsc.load_gather(fe_v, [e16 * nf + f])
                gs = plsc.load_gather(gd_v, [s16 * 16 + f])
                gdd = plsc.load_gather(gd_v, [d16 * 16 + f])
                c = jnp.where((fef > gs) & (fef > gdd), 1.0, 0.0)
                out.append(jnp.maximum(accs[f], c))
            return tuple(out)
        accs = lax.fori_loop(0, epg // 16, cyc_body,
                             tuple(jnp.zeros((16,), jnp.float32) for _ in range(nf)))
        indv = jnp.zeros((16,), jnp.float32)
        for f in range(nf):
            indv = jnp.where(iota == f, jnp.max(accs[f], axis=0), indv)

        plsc.store_scatter(out_v, [jnp.bitwise_xor(iota, nf)], mean16)
        plsc.store_scatter(out_v, [iota + 16], indv)
        plsc.store_scatter(out_v, [iota + 32], jnp.zeros((16,), jnp.float32))
        plsc.store_scatter(out_v, [iota + 48], jnp.zeros((16,), jnp.float32))
        pltpu.sync_copy(out_v, out_hbm.at[g])

    for k in range((b + nw - 1) // nw):
        g = wid + k * nw
        if (k + 1) * nw <= b:
            do_graph(g)
        else:
            @pl.when(g < b)
            def _():
                do_graph(g)


def _final_kernel(mean_ref, ind_ref, dsw_ref, dsb_ref, d1w_ref, d1b_ref,
                  ow1_ref, ob1_ref, ow2_ref, ob2_ref, bng_ref, bnb_ref,
                  out_ref):
    bf = lambda a: a.astype(jnp.bfloat16).astype(jnp.float32)
    mean32 = mean_ref[...]
    frac = jnp.mean(ind_ref[...], axis=1, keepdims=True)
    colsum = jnp.sum(bf(d1w_ref[...]), axis=0, keepdims=True)
    x0g = (jnp.dot(bf(mean32), bf(dsw_ref[...]), precision=_HI,
                   preferred_element_type=jnp.float32) + dsb_ref[0]
           + frac * colsum + d1b_ref[0])
    mu = jnp.mean(x0g, axis=0, keepdims=True)
    var = jnp.mean((x0g - mu) ** 2, axis=0, keepdims=True)
    xb = (x0g - mu) / jnp.sqrt(var + 1e-5) * bng_ref[0] + bnb_ref[0]
    h = jnp.maximum(jnp.dot(bf(xb), bf(ow1_ref[...]), precision=_HI,
                            preferred_element_type=jnp.float32) + ob1_ref[0], 0.0)
    out_ref[...] = jnp.dot(bf(h), bf(ow2_ref[...]), precision=_HI,
                           preferred_element_type=jnp.float32) + ob2_ref[0]


def kernel(x, edge_index, vertex_slices, edge_slices, batch, pos,
           fil_W1, fil_b1, fil_W2, fil_b2,
           efil_W1, efil_b1, efil_W2, efil_b2,
           ds0_W, ds0_b, ds1_W, ds1_b,
           out_W1, out_b1, out_W2, out_b2, bn_g, bn_b):
    n, d = x.shape
    e = edge_index.shape[1]
    b = vertex_slices.shape[0] - 1
    npg = n // b
    epg = e // b
    h = fil_W1.shape[1]
    nf = fil_W2.shape[1]
    out_dim = ds0_W.shape[1]

    x3 = x.reshape(b, npg, d)
    pos3 = jnp.pad(pos, ((0, 0), (0, 1))).reshape(b, npg, 4)
    offs = (jnp.arange(b, dtype=jnp.int32) * npg)[:, None]
    srcl = (edge_index[0].reshape(b, epg) - offs).reshape(b, 1, epg)
    dstl = (edge_index[1].reshape(b, epg) - offs).reshape(b, 1, epg)

    r2 = lambda a: a.reshape(1, -1)
    full = lambda shape: pl.BlockSpec(shape, lambda g: (0,) * len(shape))

    fe, am = pl.pallas_call(
        _graph_kernel,
        grid=(b,),
        compiler_params=pltpu.CompilerParams(fuse_transposed_lhs_in_matmul=True),
        in_specs=[
            pl.BlockSpec((1, npg, d), lambda g: (g, 0, 0)),
            pl.BlockSpec((1, npg, 4), lambda g: (g, 0, 0)),
            pl.BlockSpec((1, 1, epg), lambda g: (g, 0, 0)),
            pl.BlockSpec((1, 1, epg), lambda g: (g, 0, 0)),
            full((d, h)), full((1, h)), full((h, nf)), full((1, nf)),
            full((h, d)), full((h, 1)), full((h, 1)),
            full((nf, h)), full((nf, 1)),
        ],
        out_specs=[
            pl.BlockSpec((1, epg, nf), lambda g: (g, 0, 0)),
            pl.BlockSpec((1, 1, nf), lambda g: (g, 0, 0)),
        ],
        out_shape=[
            jax.ShapeDtypeStruct((b, epg, nf), jnp.float32),
            jax.ShapeDtypeStruct((b, 1, nf), jnp.float32),
        ],
    )(x3, pos3, srcl, dstl,
      fil_W1, r2(fil_b1), fil_W2, r2(fil_b2),
      efil_W1[:d].T, efil_W1[d].reshape(h, 1), efil_b1.reshape(h, 1),
      efil_W2.T, efil_b2.reshape(nf, 1))

    sc_out = pl.kernel(
        _sc_body,
        out_type=jax.ShapeDtypeStruct((b, 64), jnp.float32),
        mesh=plsc.VectorSubcoreMesh(core_axis_name="c", subcore_axis_name="s"),
        compiler_params=pltpu.CompilerParams(needs_layout_passes=False),
        scratch_types=[
            pltpu.VMEM((epg * nf,), jnp.float32),
            pltpu.VMEM((epg,), jnp.int32),
            pltpu.VMEM((epg,), jnp.int32),
            pltpu.VMEM((npg * 16,), jnp.float32),
            pltpu.VMEM((64,), jnp.float32),
        ],
    )(fe.reshape(b, epg * nf), srcl.reshape(b, epg), dstl.reshape(b, epg))

    dm = sc_out[:, 0:nf]
    gm = sc_out[:, nf:2 * nf]
    ind = sc_out[:, 2 * nf:3 * nf]
    zeros = jnp.zeros((b, nf), jnp.float32)
    mean32 = jnp.stack([zeros, dm, gm, am.reshape(b, nf)],
                       axis=-1).reshape(b, 4 * nf)

    out = pl.pallas_call(
        _final_kernel,
        in_specs=[
            pl.BlockSpec((b, 4 * nf), lambda: (0, 0)),
            pl.BlockSpec((b, nf), lambda: (0, 0)),
            pl.BlockSpec((4 * nf, out_dim), lambda: (0, 0)),
            pl.BlockSpec((1, out_dim), lambda: (0, 0)),
            pl.BlockSpec((4, out_dim), lambda: (0, 0)),
            pl.BlockSpec((1, out_dim), lambda: (0, 0)),
            pl.BlockSpec((out_dim, out_dim), lambda: (0, 0)),
            pl.BlockSpec((1, out_dim), lambda: (0, 0)),
            pl.BlockSpec((out_dim, out_dim), lambda: (0, 0)),
            pl.BlockSpec((1, out_dim), lambda: (0, 0)),
            pl.BlockSpec((1, out_dim), lambda: (0, 0)),
            pl.BlockSpec((1, out_dim), lambda: (0, 0)),
        ],
        out_specs=pl.BlockSpec((b, out_dim), lambda: (0, 0)),
        out_shape=jax.ShapeDtypeStruct((b, out_dim), jnp.float32),
    )(mean32, ind,
      ds0_W, r2(ds0_b), ds1_W, r2(ds1_b),
      out_W1, r2(out_b1), out_W2, r2(out_b2), r2(bn_g), r2(bn_b))
    return out


# speedup vs baseline: 1.1517x; 1.1517x over previous
import jax
import jax.numpy as jnp
from jax import lax
from jax.experimental import pallas as pl
from jax.experimental.pallas import tpu as pltpu, tpu_sc as plsc

_HI = jax.lax.Precision.HIGHEST


def _graph_kernel(x_ref, pos_ref, srcl_ref, dstl_ref,
                  fw1_ref, fb1_ref, fw2_ref, fb2_ref,
                  ew1t_ref, ewlast_ref, eb1_ref, ew2t_ref, eb2_ref,
                  fe_ref, am_ref):
    npg = x_ref.shape[1]
    epg = srcl_ref.shape[2]
    nf = fw2_ref.shape[1]
    xg = x_ref[0]
    posg = pos_ref[0]
    srow = srcl_ref[0]
    drow = dstl_ref[0]

    bf = lambda a: a.astype(jnp.bfloat16).astype(jnp.float32)

    h0 = jnp.maximum(jnp.dot(bf(xg), bf(fw1_ref[...]), precision=_HI,
                             preferred_element_type=jnp.float32) + fb1_ref[0], 0.0)
    fv = jax.nn.sigmoid(jnp.dot(bf(h0), bf(fw2_ref[...]), precision=_HI,
                                preferred_element_type=jnp.float32) + fb2_ref[0])

    rows = jax.lax.broadcasted_iota(jnp.int32, (npg, epg), 0)
    ohs = (rows == srow).astype(jnp.float32)
    ohd = (rows == drow).astype(jnp.float32)

    dn = (((0,), (0,)), ((), ()))
    xsum = jax.lax.dot_general(xg, ohs + ohd, dn, precision=_HI,
                               preferred_element_type=jnp.float32)
    ps = jax.lax.dot_general(posg, ohs, dn, precision=_HI,
                             preferred_element_type=jnp.float32)
    pd = jax.lax.dot_general(posg, ohd, dn, precision=_HI,
                             preferred_element_type=jnp.float32)
    d2 = jnp.sum((ps - pd) ** 2, axis=0, keepdims=True)
    dist = jnp.sqrt(d2)

    h1 = jnp.maximum(jnp.dot(bf(ew1t_ref[...]), bf(xsum), precision=_HI,
                             preferred_element_type=jnp.float32)
                     + bf(ewlast_ref[...]) * bf(dist) + eb1_ref[...], 0.0)
    fe = jax.nn.sigmoid(jnp.dot(bf(ew2t_ref[...]), bf(h1), precision=_HI,
                                preferred_element_type=jnp.float32) + eb2_ref[...])

    fe_ref[0] = fe
    am_ref[0, 0] = jnp.mean(fv, axis=0)


def _sc_body(fe_hbm, src_hbm, dst_hbm, out_hbm, fe_v, src_v, dst_v, gd_v, out_v):
    b = fe_hbm.shape[0]
    nf = 8
    epg = fe_hbm.shape[1] // nf
    npg = gd_v.shape[0] // 16
    nw = 32
    wid = lax.axis_index("s") * 2 + lax.axis_index("c")
    iota = lax.iota(jnp.int32, 16)
    lane_lt8 = iota < nf
    coloff = jnp.bitwise_and(iota, nf - 1) * epg
    init_gd = jnp.where(lane_lt8, jnp.inf, -jnp.inf)

    def splat(v):
        return jnp.full((16,), v, jnp.int32)

    def do_graph(g):
        pltpu.sync_copy(fe_hbm.at[g], fe_v)
        pltpu.sync_copy(src_hbm.at[g], src_v)
        pltpu.sync_copy(dst_hbm.at[g], dst_v)

        def init_row(i, carry):
            plsc.store_scatter(gd_v, [splat(i * 16) + iota], init_gd)
            return carry
        lax.fori_loop(0, npg, init_row, 0)

        def edge_body(k, carry):
            e16 = k * 16 + iota
            s16 = plsc.load_gather(src_v, [e16])
            d16 = plsc.load_gather(dst_v, [e16])
            for j in range(16):
                fe16 = plsc.load_gather(fe_v, [splat(k * 16 + j) + coloff])
                for node in (s16[j], d16[j]):
                    rows = splat(node * 16) + iota
                    cur = plsc.load_gather(gd_v, [rows])
                    new = jnp.where(lane_lt8, jnp.minimum(cur, fe16),
                                    jnp.maximum(cur, fe16))
                    plsc.store_scatter(gd_v, [rows], new)
            return carry
        lax.fori_loop(0, epg // 16, edge_body, 0)

        def fix_row(i, acc):
            rows = splat(i * 16) + iota
            v = plsc.load_gather(gd_v, [rows])
            v = jnp.where(jnp.abs(v) < jnp.inf, v, 1.0)
            plsc.store_scatter(gd_v, [rows], v)
            return acc + v
        sums = lax.fori_loop(0, npg, fix_row, jnp.zeros((16,), jnp.float32))
        mean16 = sums / jnp.float32(npg)

        def cyc_body(k, accs):
            e16 = k * 16 + iota
            s16 = plsc.load_gather(src_v, [e16])
            d16 = plsc.load_gather(dst_v, [e16])
            out = []
            for f in range(nf):
                fef = plsc.load_gather(fe_v, [e16 + f * epg])
                gs = plsc.load_gather(gd_v, [s16 * 16 + f])
                gdd = plsc.load_gather(gd_v, [d16 * 16 + f])
                c = jnp.where((fef > gs) & (fef > gdd), 1.0, 0.0)
                out.append(jnp.maximum(accs[f], c))
            return tuple(out)
        accs = lax.fori_loop(0, epg // 16, cyc_body,
                             tuple(jnp.zeros((16,), jnp.float32) for _ in range(nf)))
        indv = jnp.zeros((16,), jnp.float32)
        for f in range(nf):
            indv = jnp.where(iota == f, jnp.max(accs[f], axis=0), indv)

        plsc.store_scatter(out_v, [jnp.bitwise_xor(iota, nf)], mean16)
        plsc.store_scatter(out_v, [iota + 16], indv)
        plsc.store_scatter(out_v, [iota + 32], jnp.zeros((16,), jnp.float32))
        plsc.store_scatter(out_v, [iota + 48], jnp.zeros((16,), jnp.float32))
        pltpu.sync_copy(out_v, out_hbm.at[g])

    for k in range((b + nw - 1) // nw):
        g = wid + k * nw
        if (k + 1) * nw <= b:
            do_graph(g)
        else:
            @pl.when(g < b)
            def _():
                do_graph(g)


def _final_kernel(mean_ref, ind_ref, dsw_ref, dsb_ref, d1w_ref, d1b_ref,
                  ow1_ref, ob1_ref, ow2_ref, ob2_ref, bng_ref, bnb_ref,
                  out_ref):
    bf = lambda a: a.astype(jnp.bfloat16).astype(jnp.float32)
    mean32 = mean_ref[...]
    frac = jnp.mean(ind_ref[...], axis=1, keepdims=True)
    colsum = jnp.sum(bf(d1w_ref[...]), axis=0, keepdims=True)
    x0g = (jnp.dot(bf(mean32), bf(dsw_ref[...]), precision=_HI,
                   preferred_element_type=jnp.float32) + dsb_ref[0]
           + frac * colsum + d1b_ref[0])
    mu = jnp.mean(x0g, axis=0, keepdims=True)
    var = jnp.mean((x0g - mu) ** 2, axis=0, keepdims=True)
    xb = (x0g - mu) / jnp.sqrt(var + 1e-5) * bng_ref[0] + bnb_ref[0]
    h = jnp.maximum(jnp.dot(bf(xb), bf(ow1_ref[...]), precision=_HI,
                            preferred_element_type=jnp.float32) + ob1_ref[0], 0.0)
    out_ref[...] = jnp.dot(bf(h), bf(ow2_ref[...]), precision=_HI,
                           preferred_element_type=jnp.float32) + ob2_ref[0]


def kernel(x, edge_index, vertex_slices, edge_slices, batch, pos,
           fil_W1, fil_b1, fil_W2, fil_b2,
           efil_W1, efil_b1, efil_W2, efil_b2,
           ds0_W, ds0_b, ds1_W, ds1_b,
           out_W1, out_b1, out_W2, out_b2, bn_g, bn_b):
    n, d = x.shape
    e = edge_index.shape[1]
    b = vertex_slices.shape[0] - 1
    npg = n // b
    epg = e // b
    h = fil_W1.shape[1]
    nf = fil_W2.shape[1]
    out_dim = ds0_W.shape[1]

    x3 = x.reshape(b, npg, d)
    pos3 = jnp.pad(pos, ((0, 0), (0, 1))).reshape(b, npg, 4)
    offs = (jnp.arange(b, dtype=jnp.int32) * npg)[:, None]
    srcl = (edge_index[0].reshape(b, epg) - offs).reshape(b, 1, epg)
    dstl = (edge_index[1].reshape(b, epg) - offs).reshape(b, 1, epg)

    r2 = lambda a: a.reshape(1, -1)
    full = lambda shape: pl.BlockSpec(shape, lambda g: (0,) * len(shape))

    fe, am = pl.pallas_call(
        _graph_kernel,
        grid=(b,),
        compiler_params=pltpu.CompilerParams(fuse_transposed_lhs_in_matmul=True),
        in_specs=[
            pl.BlockSpec((1, npg, d), lambda g: (g, 0, 0)),
            pl.BlockSpec((1, npg, 4), lambda g: (g, 0, 0)),
            pl.BlockSpec((1, 1, epg), lambda g: (g, 0, 0)),
            pl.BlockSpec((1, 1, epg), lambda g: (g, 0, 0)),
            full((d, h)), full((1, h)), full((h, nf)), full((1, nf)),
            full((h, d)), full((h, 1)), full((h, 1)),
            full((nf, h)), full((nf, 1)),
        ],
        out_specs=[
            pl.BlockSpec((1, nf, epg), lambda g: (g, 0, 0)),
            pl.BlockSpec((1, 1, nf), lambda g: (g, 0, 0)),
        ],
        out_shape=[
            jax.ShapeDtypeStruct((b, nf, epg), jnp.float32),
            jax.ShapeDtypeStruct((b, 1, nf), jnp.float32),
        ],
    )(x3, pos3, srcl, dstl,
      fil_W1, r2(fil_b1), fil_W2, r2(fil_b2),
      efil_W1[:d].T, efil_W1[d].reshape(h, 1), efil_b1.reshape(h, 1),
      efil_W2.T, efil_b2.reshape(nf, 1))

    sc_out = pl.kernel(
        _sc_body,
        out_type=jax.ShapeDtypeStruct((b, 64), jnp.float32),
        mesh=plsc.VectorSubcoreMesh(core_axis_name="c", subcore_axis_name="s"),
        compiler_params=pltpu.CompilerParams(needs_layout_passes=False),
        scratch_types=[
            pltpu.VMEM((epg * nf,), jnp.float32),
            pltpu.VMEM((epg,), jnp.int32),
            pltpu.VMEM((epg,), jnp.int32),
            pltpu.VMEM((npg * 16,), jnp.float32),
            pltpu.VMEM((64,), jnp.float32),
        ],
    )(fe.reshape(b, epg * nf), srcl.reshape(b, epg), dstl.reshape(b, epg))

    dm = sc_out[:, 0:nf]
    gm = sc_out[:, nf:2 * nf]
    ind = sc_out[:, 2 * nf:3 * nf]
    zeros = jnp.zeros((b, nf), jnp.float32)
    mean32 = jnp.stack([zeros, dm, gm, am.reshape(b, nf)],
                       axis=-1).reshape(b, 4 * nf)

    out = pl.pallas_call(
        _final_kernel,
        in_specs=[
            pl.BlockSpec((b, 4 * nf), lambda: (0, 0)),
            pl.BlockSpec((b, nf), lambda: (0, 0)),
            pl.BlockSpec((4 * nf, out_dim), lambda: (0, 0)),
            pl.BlockSpec((1, out_dim), lambda: (0, 0)),
            pl.BlockSpec((4, out_dim), lambda: (0, 0)),
            pl.BlockSpec((1, out_dim), lambda: (0, 0)),
            pl.BlockSpec((out_dim, out_dim), lambda: (0, 0)),
            pl.BlockSpec((1, out_dim), lambda: (0, 0)),
            pl.BlockSpec((out_dim, out_dim), lambda: (0, 0)),
            pl.BlockSpec((1, out_dim), lambda: (0, 0)),
            pl.BlockSpec((1, out_dim), lambda: (0, 0)),
            pl.BlockSpec((1, out_dim), lambda: (0, 0)),
        ],
        out_specs=pl.BlockSpec((b, out_dim), lambda: (0, 0)),
        out_shape=jax.ShapeDtypeStruct((b, out_dim), jnp.float32),
    )(mean32, ind,
      ds0_W, r2(ds0_b), ds1_W, r2(ds1_b),
      out_W1, r2(out_b1), out_W2, r2(out_b2), r2(bn_g), r2(bn_b))
    return out
